# baseline (speedup 1.0000x reference)
import jax
import jax.numpy as jnp
from jax.experimental import pallas as pl


def _noop_body(x_ref, o_ref):
    o_ref[...] = x_ref[...]


def kernel(node_ids, messages, memory, W_ih, W_hh, b_ih, b_hh):
    messages = pl.pallas_call(
        _noop_body,
        out_shape=jax.ShapeDtypeStruct(messages.shape, messages.dtype),
    )(messages)
    gi = messages @ W_ih.T + b_ih
    current = jnp.take(memory, node_ids, axis=0)
    gh = current @ W_hh.T + b_hh
    i_r, i_z, i_n = jnp.split(gi, 3, axis=-1)
    h_r, h_z, h_n = jnp.split(gh, 3, axis=-1)
    r = jax.nn.sigmoid(i_r + h_r)
    z = jax.nn.sigmoid(i_z + h_z)
    n = jnp.tanh(i_n + r * h_n)
    new_state = (1.0 - z) * n + z * current
    return memory.at[node_ids].set(new_state)


# speedup vs baseline: 1.3473x; 1.3473x over previous
import functools

import jax
import jax.numpy as jnp
from jax import lax
from jax.experimental import pallas as pl
from jax.experimental.pallas import tpu as pltpu
from jax.experimental.pallas import tpu_sc as plsc

_INFO = plsc.get_sparse_core_info()
_NC, _NS, _L = _INFO.num_cores, _INFO.num_subcores, _INFO.num_lanes
_NW = _NC * _NS


def _wid():
    return lax.axis_index("s") * _NC + lax.axis_index("c")


def _mesh():
    return plsc.VectorSubcoreMesh(core_axis_name="c", subcore_axis_name="s")


def _make_gather(n, d, b):
    bw = b // _NW

    @functools.partial(
        pl.kernel,
        mesh=_mesh(),
        out_type=jax.ShapeDtypeStruct((b, d), jnp.float32),
        scratch_types=[
            pltpu.VMEM((bw,), jnp.int32),
            pltpu.VMEM((bw, d), jnp.float32),
            pltpu.SemaphoreType.DMA,
        ],
        compiler_params=pltpu.CompilerParams(use_tc_tiling_on_sc=False),
    )
    def k(table, idx, out, idx_v, rows_v, sem):
        base = _wid() * bw
        pltpu.sync_copy(idx.at[pl.ds(base, bw)], idx_v)
        pltpu.async_copy(table.at[idx_v], rows_v, sem).wait()
        pltpu.sync_copy(rows_v, out.at[pl.ds(base, bw)])

    return k


def _gru_body(x_ref, h_ref, wih_ref, whh_ref, bih_ref, bhh_ref, o_ref):
    x = x_ref[...]
    h = h_ref[...]
    dn = (((1,), (1,)), ((), ()))
    gi = lax.dot_general(x, wih_ref[...], dn, preferred_element_type=jnp.float32)
    gi = gi + bih_ref[...]
    gh = lax.dot_general(h, whh_ref[...], dn, preferred_element_type=jnp.float32)
    gh = gh + bhh_ref[...]
    d = x.shape[1]
    r = jax.nn.sigmoid(gi[:, :d] + gh[:, :d])
    z = jax.nn.sigmoid(gi[:, d : 2 * d] + gh[:, d : 2 * d])
    n = jnp.tanh(gi[:, 2 * d :] + r * gh[:, 2 * d :])
    o_ref[...] = (1.0 - z) * n + z * h


def _make_gru(b, d, blk):
    grid = b // blk
    return pl.pallas_call(
        _gru_body,
        grid=(grid,),
        in_specs=[
            pl.BlockSpec((blk, d), lambda i: (i, 0)),
            pl.BlockSpec((blk, d), lambda i: (i, 0)),
            pl.BlockSpec((3 * d, d), lambda i: (0, 0)),
            pl.BlockSpec((3 * d, d), lambda i: (0, 0)),
            pl.BlockSpec((1, 3 * d), lambda i: (0, 0)),
            pl.BlockSpec((1, 3 * d), lambda i: (0, 0)),
        ],
        out_specs=pl.BlockSpec((blk, d), lambda i: (i, 0)),
        out_shape=jax.ShapeDtypeStruct((b, d), jnp.float32),
    )


def _make_tag(n, b, tagsh):
    tagn = tagsh * _NW

    @functools.partial(
        pl.kernel,
        mesh=_mesh(),
        out_type=jax.ShapeDtypeStruct((tagn,), jnp.int32),
        scratch_types=[
            pltpu.VMEM((b,), jnp.int32),
            pltpu.VMEM((tagsh,), jnp.int32),
        ],
        compiler_params=pltpu.CompilerParams(needs_layout_passes=False),
    )
    def k(idx, tag, ids_v, t_v):
        lo = _wid() * tagsh
        pltpu.sync_copy(idx, ids_v)
        lane = lax.iota(jnp.int32, _L)
        nxt_idx = jnp.minimum(lane + 1, _L - 1)

        def step(j, carry):
            v = ids_v[pl.ds(j * _L, _L)]
            fused = v * _L + lane
            pos = lane + j * _L
            sk, sv = plsc.sort_key_val(fused, pos)
            sid = lax.shift_right_logical(sk, 4)
            nxt = lax.gather(
                sid, nxt_idx[:, None],
                lax.GatherDimensionNumbers(
                    offset_dims=(), collapsed_slice_dims=(0,),
                    start_index_map=(0,)),
                slice_sizes=(1,),
                mode=lax.GatherScatterMode.PROMISE_IN_BOUNDS)
            is_last = (sid != nxt) | (lane == _L - 1)
            inr = (sid >= lo) & (sid < lo + tagsh)
            plsc.store_scatter(t_v, [sid - lo], sv, mask=is_last & inr)
            return carry

        lax.fori_loop(0, b // _L, step, 0)
        pltpu.sync_copy(t_v, tag.at[pl.ds(lo, tagsh)])

    return k


def _make_copy_scatter(n, d, b, tagsh, ch, sch):
    rw = n // _NW
    n_chunks = (rw + ch - 1) // ch

    @functools.partial(
        pl.kernel,
        mesh=_mesh(),
        out_type=jax.ShapeDtypeStruct((n, d), jnp.float32),
        compiler_params=pltpu.CompilerParams(
            use_tc_tiling_on_sc=False, needs_layout_passes=False
        ),
    )
    def k(table, idx, tag, news, out):
        w = _wid()
        r0 = w * rw

        assert rw % ch == 0 and n_chunks % 2 == 0
        n_pairs = n_chunks // 2

        def phase_a(buf0, buf1, rs0, rs1, ws0, ws1):
            def rd(c, buf, sem):
                pltpu.make_async_copy(table.at[pl.ds(r0 + c * ch, ch)], buf, sem).start()

            def rd_wait(c, buf, sem):
                pltpu.make_async_copy(table.at[pl.ds(r0 + c * ch, ch)], buf, sem).wait()

            def wr(c, buf, sem):
                pltpu.make_async_copy(buf, out.at[pl.ds(r0 + c * ch, ch)], sem).start()

            def wr_wait(c, buf, sem):
                pltpu.make_async_copy(buf, out.at[pl.ds(r0 + c * ch, ch)], sem).wait()

            rd(0, buf0, rs0)
            rd(1, buf1, rs1)

            def pair(g, carry):
                c0 = 2 * g
                rd_wait(c0, buf0, rs0)
                wr(c0, buf0, ws0)
                rd_wait(c0 + 1, buf1, rs1)
                wr(c0 + 1, buf1, ws1)
                wr_wait(c0, buf0, ws0)
                rd(c0 + 2, buf0, rs0)
                wr_wait(c0 + 1, buf1, ws1)
                rd(c0 + 3, buf1, rs1)
                return carry

            lax.fori_loop(0, n_pairs - 1, pair, 0)
            c0 = 2 * (n_pairs - 1)
            rd_wait(c0, buf0, rs0)
            wr(c0, buf0, ws0)
            rd_wait(c0 + 1, buf1, rs1)
            wr(c0 + 1, buf1, ws1)
            wr_wait(c0, buf0, ws0)
            wr_wait(c0 + 1, buf1, ws1)

        pl.run_scoped(
            phase_a,
            pltpu.VMEM((ch, d), jnp.float32),
            pltpu.VMEM((ch, d), jnp.float32),
            pltpu.SemaphoreType.DMA,
            pltpu.SemaphoreType.DMA,
            pltpu.SemaphoreType.DMA,
            pltpu.SemaphoreType.DMA,
        )

        def phase_b(ids_v, cidx, idxb, wv, vals, sem2):
            hi = r0 + rw
            pltpu.sync_copy(idx, ids_v)
            ids0 = ids_v[pl.ds(0, _L)][0]

            def cstep(j, off):
                v = ids_v[pl.ds(j * _L, _L)]
                m = (v >= r0) & (v < hi)
                plsc.store_compressed(cidx.at[pl.ds(off, _L)], v, mask=m)
                cnt = jnp.max(plsc.all_reduce_population_count(m))
                return off + cnt

            kcnt = lax.fori_loop(0, b // _L, cstep, 0)
            rounds = (kcnt + sch - 1) // sch
            end = rounds * sch

            lane = lax.iota(jnp.int32, _L)
            a0 = (kcnt // _L) * _L
            cur = cidx[pl.ds(a0, _L)]
            cidx[pl.ds(a0, _L)] = jnp.where(lane < (kcnt - a0), cur, ids0)

            def pad_body(off):
                cidx[pl.ds(off, _L)] = jnp.full((_L,), ids0, jnp.int32)
                return off + _L

            lax.while_loop(lambda off: off < end, pad_body, a0 + _L)

            def sstep(c, carry):
                def mv(t, carry2):
                    idxb[pl.ds(t * _L, _L)] = cidx[pl.ds(c * sch + t * _L, _L)]
                    return carry2

                lax.fori_loop(0, sch // _L, mv, 0)
                pltpu.async_copy(tag.at[idxb], wv, sem2).wait()
                pltpu.async_copy(news.at[wv], vals, sem2).wait()
                pltpu.async_copy(vals, out.at[idxb], sem2).wait()
                return carry

            lax.fori_loop(0, rounds, sstep, 0)

        pl.run_scoped(
            phase_b,
            pltpu.VMEM((b,), jnp.int32),
            pltpu.VMEM((b + _L,), jnp.int32),
            pltpu.VMEM((sch,), jnp.int32),
            pltpu.VMEM((sch,), jnp.int32),
            pltpu.VMEM((sch, d), jnp.float32),
            pltpu.SemaphoreType.DMA,
        )

    return k


def kernel(node_ids, messages, memory, W_ih, W_hh, b_ih, b_hh):
    n, d = memory.shape
    b = node_ids.shape[0]
    ids = node_ids.astype(jnp.int32)

    current = _make_gather(n, d, b)(memory, ids)
    new_state = _make_gru(b, d, 2048)(
        messages, current, W_ih, W_hh,
        b_ih.reshape(1, 3 * d), b_hh.reshape(1, 3 * d),
    )
    tagsh = ((n + _NW - 1) // _NW + 7) // 8 * 8
    tag = _make_tag(n, b, tagsh)(ids)
    out = _make_copy_scatter(n, d, b, tagsh, 625, 512)(memory, ids, tag, new_state)
    return out
